# baseline (speedup 1.0000x reference)
import jax
import jax.numpy as jnp
from jax import lax
from jax.experimental import pallas as pl
from jax.experimental.pallas import tpu as pltpu
from jax.experimental.pallas import tpu_sc as plsc

NC, NS = 2, 16
NW = NC * NS
B, P, D = 1024, 77, 768
SEQ_PER_W = B // NW
LANES = 16
G = D // LANES


def _body(idx_hbm, table_hbm, pos_hbm, out_hbm, idx_v, pos_v, rows_v, sem):
    wid = lax.axis_index("s") * NC + lax.axis_index("c")
    seq0 = wid * SEQ_PER_W

    pltpu.sync_copy(idx_hbm.at[pl.ds(seq0, SEQ_PER_W)], idx_v)
    pltpu.sync_copy(pos_hbm, pos_v)

    def step(s, carry):
        pltpu.async_copy(table_hbm.at[idx_v.at[s]], rows_v, sem).wait()

        def add_row(r, c):
            for g in range(G):
                sl = pl.ds(g * LANES, LANES)
                plsc.addupdate(rows_v.at[r, sl], pos_v[r, sl])
            return c

        lax.fori_loop(0, P, add_row, 0)

        pltpu.sync_copy(rows_v, out_hbm.at[pl.ds((seq0 + s) * P, P)])
        return carry

    lax.fori_loop(0, SEQ_PER_W, step, 0)


def kernel(input_tokens, token_table, position_table):
    idx = input_tokens.astype(jnp.int32)
    mesh = plsc.VectorSubcoreMesh(
        core_axis_name="c", subcore_axis_name="s", num_cores=NC, num_subcores=NS
    )
    out = pl.kernel(
        _body,
        out_type=jax.ShapeDtypeStruct((B * P, D), jnp.float32),
        mesh=mesh,
        compiler_params=pltpu.CompilerParams(use_tc_tiling_on_sc=False),
        scratch_types=[
            pltpu.VMEM((SEQ_PER_W, P), jnp.int32),
            pltpu.VMEM((P, D), jnp.float32),
            pltpu.VMEM((P, D), jnp.float32),
            pltpu.SemaphoreType.DMA,
        ],
    )(idx, token_table, position_table)
    return out.reshape(B, P, D)
